# Initial kernel scaffold; baseline (speedup 1.0000x reference)
import jax
import jax.numpy as jnp
from jax.experimental import pallas as pl


def kernel(unary, trans, lengths):
    raise NotImplementedError("write your pallas kernel here")


# speedup vs baseline: 6.8229x; 6.8229x over previous
import jax
import jax.numpy as jnp
from jax.experimental import pallas as pl
from jax.experimental.pallas import tpu as pltpu

_START = 126
_END = 127
_NEG = -10000.0


def _viterbi_body(u_ref, tt_ref, t_ref, tend_ref, path_ref, score_ref, av_ref):
    L, N = u_ref.shape
    col_iota = jax.lax.broadcasted_iota(jnp.int32, (N, 1), 0)
    a_col0 = jnp.where(col_iota == _START, 0.0, _NEG).astype(jnp.float32)

    def fwd(t, a_col):
        scores = tt_ref[...] + a_col
        vit = jnp.max(scores, axis=0, keepdims=True)
        a_row = vit + u_ref[pl.ds(t, 1), :]
        av_ref[pl.ds(t, 1), :] = a_row
        blk = jnp.broadcast_to(a_row, (8, N))
        return jnp.transpose(blk)[:, 0:1]

    jax.lax.fori_loop(0, L, fwd, a_col0)

    lane = jax.lax.broadcasted_iota(jnp.int32, (1, N), 1)
    terminal = av_ref[pl.ds(L - 1, 1), :] + tend_ref[...]
    m = jnp.max(terminal)
    b0 = jnp.min(jnp.where(terminal == m, lane, N))
    score_ref[...] = jnp.broadcast_to(m, (1, 1))

    def bwd(k, carry):
        tag, row = carry
        t = L - 1 - k
        row = jnp.where(lane == (t % 128), tag, row)

        @pl.when(t % 128 == 0)
        def _store():
            path_ref[pl.ds(t // 128, 1), :] = row

        tp = jnp.maximum(t - 1, 0)
        srow = av_ref[pl.ds(tp, 1), :] + t_ref[pl.ds(tag, 1), :]
        m2 = jnp.max(srow)
        nxt = jnp.min(jnp.where(srow == m2, lane, N))
        return nxt, row

    jax.lax.fori_loop(0, L, bwd, (b0, jnp.zeros((1, N), jnp.int32)))


@jax.jit
def kernel(unary, trans, lengths):
    u = unary[:, 0, :]
    t_mat = trans[0]
    L, N = u.shape
    tt = t_mat.T
    tend = t_mat[_END][None, :]
    path2d, score = pl.pallas_call(
        _viterbi_body,
        out_shape=(
            jax.ShapeDtypeStruct((L // 128, 128), jnp.int32),
            jax.ShapeDtypeStruct((1, 1), jnp.float32),
        ),
        scratch_shapes=[pltpu.VMEM((L, N), jnp.float32)],
    )(u, tt, t_mat, tend)
    path = path2d.reshape(L)[:, None]
    return path, score[0, 0]


# speedup vs baseline: 15.3043x; 2.2431x over previous
import functools

import jax
import jax.numpy as jnp
from jax import lax
from jax.experimental import pallas as pl
from jax.experimental.pallas import tpu as pltpu
from jax.experimental.pallas import tpu_sc as plsc

_START = 126
_END = 127
_NEG = -10000.0


def _viterbi_fwd_body(u_ref, tt_ref, tend_ref, bp_ref, b0_ref, score_ref):
    L, N = u_ref.shape
    col_iota = jax.lax.broadcasted_iota(jnp.int32, (N, 1), 0)
    idx_col = jax.lax.broadcasted_iota(jnp.int32, (N, N), 0)
    a_col0 = jnp.where(col_iota == _START, 0.0, _NEG).astype(jnp.float32)

    def fwd(t, a_col):
        scores = tt_ref[...] + a_col
        vit = jnp.max(scores, axis=0, keepdims=True)
        best = jnp.min(
            jnp.where(scores == vit, idx_col, N), axis=0, keepdims=True
        )
        bp_ref[pl.ds(t, 1), :] = best
        a_row = vit + u_ref[pl.ds(t, 1), :]
        blk = jnp.broadcast_to(a_row, (8, N))
        return jnp.transpose(blk)[:, 0:1]

    a_fin = jax.lax.fori_loop(0, L, fwd, a_col0)

    lane = jax.lax.broadcasted_iota(jnp.int32, (1, N), 1)
    terminal = jnp.transpose(jnp.broadcast_to(a_fin, (N, 8)))[0:1, :] \
        + tend_ref[...]
    m = jnp.max(terminal)
    b0 = jnp.min(jnp.where(terminal == m, lane, N))
    score_ref[...] = jnp.broadcast_to(m, (1, 1))
    b0_ref[...] = jnp.broadcast_to(b0, (1, N))


def _make_chase(L):
    CH = 512
    mesh = plsc.VectorSubcoreMesh(core_axis_name="c", subcore_axis_name="s")

    @functools.partial(
        pl.kernel,
        mesh=mesh,
        out_type=jax.ShapeDtypeStruct((L,), jnp.int32),
        scratch_types=[
            pltpu.VMEM((CH * 128,), jnp.int32),
            pltpu.VMEM((L,), jnp.int32),
            pltpu.VMEM((128,), jnp.int32),
        ],
    )
    def chase(bp_hbm, b0_hbm, path_hbm, bp_buf, path_buf, b0_buf):
        wid = lax.axis_index("c") * 16 + lax.axis_index("s")

        @pl.when(wid == 0)
        def _():
            lanes = lax.iota(jnp.int32, 16)
            pltpu.sync_copy(b0_hbm, b0_buf)
            tag0 = b0_buf[pl.ds(0, 16)][0]
            vec0 = jnp.where(lanes == 15, tag0, jnp.zeros((16,), jnp.int32))

            carry = (tag0, vec0)
            for ci in range(L // CH - 1, -1, -1):
                base = ci * CH
                n_steps = CH if ci > 0 else CH - 1
                pltpu.sync_copy(
                    bp_hbm.at[pl.ds(base * 128, CH * 128)], bp_buf
                )

                def step(k, carry):
                    tg, vec = carry
                    t = base + CH - 1 - k
                    nxt = bp_buf[pl.ds((CH - 1 - k) * 128 + tg, 16)][0]
                    vec = jnp.where(lanes == (t - 1) % 16, nxt, vec)

                    @pl.when((t - 1) % 16 == 0)
                    def _flush():
                        path_buf[pl.ds(t - 1, 16)] = vec

                    return nxt, vec

                carry = jax.lax.fori_loop(0, n_steps, step, carry)

            pltpu.sync_copy(path_buf, path_hbm)

    return chase


@jax.jit
def kernel(unary, trans, lengths):
    u = unary[:, 0, :]
    t_mat = trans[0]
    L, N = u.shape
    tt = t_mat.T
    tend = t_mat[_END][None, :]
    bp, b0, score = pl.pallas_call(
        _viterbi_fwd_body,
        out_shape=(
            jax.ShapeDtypeStruct((L, N), jnp.int32),
            jax.ShapeDtypeStruct((1, N), jnp.int32),
            jax.ShapeDtypeStruct((1, 1), jnp.float32),
        ),
    )(u, tt, tend)
    path = _make_chase(L)(bp.reshape(L * N), b0[0])
    return path[:, None], score[0, 0]
